# baseline (speedup 1.0000x reference)
import functools

import jax
import jax.numpy as jnp
from jax import lax
from jax.experimental import pallas as pl
from jax.experimental.pallas import tpu as pltpu
from jax.experimental.pallas import tpu_sc as plsc

_B, _S, _D = 4, 2048, 128
_N = _B * _S
_L = 16
_NC, _NS = 2, 16
_NW = _NC * _NS
_BPW = _N // _NW
_WPB = _S // _BPW
_CH = 64
_NCH = _BPW // _CH

_mesh = plsc.VectorSubcoreMesh(core_axis_name="c", subcore_axis_name="s")


@functools.partial(
    pl.kernel,
    mesh=_mesh,
    out_type=jax.ShapeDtypeStruct((_B, _S, _D), jnp.float32),
    scratch_types=[
        pltpu.VMEM((_BPW,), jnp.int32),
        pltpu.VMEM((_BPW,), jnp.int32),
        pltpu.VMEM((2, _CH, _D), jnp.float32),
        pltpu.VMEM((2, _CH, _D), jnp.float32),
        pltpu.SemaphoreType.DMA,
        pltpu.SemaphoreType.DMA,
        pltpu.SemaphoreType.DMA,
        pltpu.SemaphoreType.DMA,
        pltpu.SemaphoreType.DMA,
        pltpu.SemaphoreType.DMA,
    ],
)
def _embed(vt_hbm, pt_hbm, ids_hbm, pids_hbm, out_hbm,
           idx_v, pidx_v, rows, prows, sv0, sv1, sp0, sp1, so0, so1):
    sv = (sv0, sv1)
    sp = (sp0, sp1)
    so = (so0, so1)
    wid = lax.axis_index("s") * _NC + lax.axis_index("c")
    brow = wid // _WPB
    scol = (wid % _WPB) * _BPW
    pltpu.sync_copy(ids_hbm.at[brow, pl.ds(scol, _BPW)], idx_v)
    pltpu.sync_copy(pids_hbm.at[brow, pl.ds(scol, _BPW)], pidx_v)

    def start_gather(c):
        b = c % 2
        cv = pltpu.async_copy(
            vt_hbm.at[idx_v.at[pl.ds(c * _CH, _CH)]], rows.at[b], sv[b])
        cp = pltpu.async_copy(
            pt_hbm.at[pidx_v.at[pl.ds(c * _CH, _CH)]], prows.at[b], sp[b])
        return cv, cp

    gathers = {0: start_gather(0)}
    out_cps = {}
    for c in range(_NCH):
        b = c % 2
        cv, cp = gathers[c]
        cv.wait()
        cp.wait()
        if c + 1 < _NCH:
            if c >= 1:
                out_cps[c - 1].wait()
            gathers[c + 1] = start_gather(c + 1)

        @plsc.parallel_loop(0, _CH, unroll=2)
        def _add(i):
            for j in range(_D // _L):
                s = pl.ds(j * _L, _L)
                rows[b, i, s] = rows[b, i, s] + prows[b, i, s]

        out_cps[c] = pltpu.async_copy(
            rows.at[b], out_hbm.at[brow, pl.ds(scol + c * _CH, _CH)], so[b])
    out_cps[_NCH - 2].wait()
    out_cps[_NCH - 1].wait()


def kernel(input_ids, position_ids, vocab_table, pos_table):
    return _embed(vocab_table, pos_table, input_ids, position_ids)


# speedup vs baseline: 1.0286x; 1.0286x over previous
import functools

import jax
import jax.numpy as jnp
from jax import lax
from jax.experimental import pallas as pl
from jax.experimental.pallas import tpu as pltpu
from jax.experimental.pallas import tpu_sc as plsc

_B, _S, _D = 4, 2048, 128
_N = _B * _S
_L = 16
_NC, _NS = 2, 16
_NW = _NC * _NS
_BPW = _N // _NW
_WPB = _S // _BPW
_CH = 64
_NCH = _BPW // _CH
_SROWS = _S // _NS

_mesh = plsc.VectorSubcoreMesh(core_axis_name="c", subcore_axis_name="s")


@functools.partial(
    pl.kernel,
    mesh=_mesh,
    out_type=jax.ShapeDtypeStruct((_B, _S, _D), jnp.float32),
    scratch_types=[
        pltpu.VMEM((_BPW,), jnp.int32),
        pltpu.VMEM((_BPW,), jnp.int32),
        pltpu.VMEM((2, _CH, _D), jnp.float32),
        pltpu.VMEM((_BPW, _D), jnp.float32),
        pltpu.VMEM_SHARED((_S, _D), jnp.float32),
        pltpu.SemaphoreType.DMA,
        pltpu.SemaphoreType.DMA,
        pltpu.SemaphoreType.DMA,
        pltpu.SemaphoreType.DMA,
        pltpu.SemaphoreType.DMA,
        pltpu.SemaphoreType.DMA,
        pltpu.SemaphoreType.DMA,
        pltpu.SemaphoreType.DMA,
        pltpu.SemaphoreType.DMA,
        pltpu.SemaphoreType.DMA,
        pltpu.SemaphoreType.DMA,
    ],
)
def _embed(vt_hbm, pt_hbm, ids_hbm, pids_hbm, out_hbm,
           idx_v, pidx_v, rows, prows, pos_sh,
           si0, si1, ss, sv0, sv1, sp0, sp1, sp2, sp3, so0, so1):
    sv = (sv0, sv1)
    sp = (sp0, sp1, sp2, sp3)
    so = (so0, so1)
    cid = lax.axis_index("c")
    sid = lax.axis_index("s")
    wid = sid * _NC + cid
    brow = wid // _WPB
    scol = (wid % _WPB) * _BPW

    cp_i = pltpu.async_copy(ids_hbm.at[brow, pl.ds(scol, _BPW)], idx_v, si0)
    cp_p = pltpu.async_copy(pids_hbm.at[brow, pl.ds(scol, _BPW)], pidx_v, si1)
    cp_s = pltpu.async_copy(
        pt_hbm.at[pl.ds(sid * _SROWS, _SROWS)],
        pos_sh.at[pl.ds(sid * _SROWS, _SROWS)], ss)

    cp_i.wait()

    def start_vgather(c):
        b = c % 2
        return pltpu.async_copy(
            vt_hbm.at[idx_v.at[pl.ds(c * _CH, _CH)]], rows.at[b], sv[b])

    vgathers = {0: start_vgather(0)}

    cp_p.wait()
    cp_s.wait()
    plsc.subcore_barrier()
    pgathers = {
        c: pltpu.async_copy(
            pos_sh.at[pidx_v.at[pl.ds(c * _CH, _CH)]],
            prows.at[pl.ds(c * _CH, _CH)], sp[c])
        for c in range(_NCH)
    }

    out_cps = {}
    for c in range(_NCH):
        b = c % 2
        vgathers[c].wait()
        if c + 1 < _NCH:
            if c >= 1:
                out_cps[c - 1].wait()
            vgathers[c + 1] = start_vgather(c + 1)
        pgathers[c].wait()

        @plsc.parallel_loop(0, _CH, unroll=2)
        def _add(i):
            for j in range(_D // _L):
                s = pl.ds(j * _L, _L)
                rows[b, i, s] = rows[b, i, s] + prows[c * _CH + i, s]

        out_cps[c] = pltpu.async_copy(
            rows.at[b], out_hbm.at[brow, pl.ds(scol + c * _CH, _CH)], so[b])
    out_cps[_NCH - 2].wait()
    out_cps[_NCH - 1].wait()


def kernel(input_ids, position_ids, vocab_table, pos_table):
    return _embed(vocab_table, pos_table, input_ids, position_ids)
